# baseline (speedup 1.0000x reference)
import functools
import math

import jax
import jax.numpy as jnp
import numpy as np
from jax import lax
from jax.experimental import pallas as pl
from jax.experimental.pallas import tpu as pltpu
from jax.experimental.pallas import tpu_sc as plsc

N = 10000
E = 320000
D = 128
DW = 16
NC, NS = 2, 16
NH = N // NC
EW = E // NS
B = 32
NB = EW // B
RB = 125
NCH = NH // RB
RN = 2000
EB = 4000


def _proj_body(x_ref, wq_ref, bq_ref, wk_ref, bk_ref, wv_ref, bv_ref,
               ws_ref, bs_ref, q_ref, k_ref, v_ref, s_ref, *, qscale):
    xb = x_ref[...]
    dot = lambda w: jnp.dot(xb, w[...], preferred_element_type=jnp.float32)
    q_ref[...] = (dot(wq_ref) + bq_ref[...]) * qscale
    k_ref[...] = dot(wk_ref) + bk_ref[...]
    v_ref[...] = dot(wv_ref) + bv_ref[...]
    s_ref[...] = dot(ws_ref) + bs_ref[...]


def _proj(x, wq, bq, wk, bk, wv, bv, ws, bs, qscale):
    n = x.shape[0]
    wspec = pl.BlockSpec((D, D), lambda i: (0, 0))
    bspec = pl.BlockSpec((1, D), lambda i: (0, 0))
    rspec = pl.BlockSpec((RN, D), lambda i: (i, 0))
    fn = pl.pallas_call(
        functools.partial(_proj_body, qscale=qscale),
        grid=(n // RN,),
        in_specs=[rspec, wspec, bspec, wspec, bspec, wspec, bspec, wspec,
                  bspec],
        out_specs=[rspec, rspec, rspec, rspec],
        out_shape=[jax.ShapeDtypeStruct((n, D), jnp.float32)] * 4,
    )
    return fn(x, wq, bq.reshape(1, D), wk, bk.reshape(1, D),
              wv, bv.reshape(1, D), ws, bs.reshape(1, D))


def _embed_body(ea_ref, we1_ref, we2_ref, e1_ref, e2_ref):
    eab = ea_ref[...]
    e1_ref[...] = jnp.dot(eab, we1_ref[...], preferred_element_type=jnp.float32)
    e2_ref[...] = jnp.dot(eab, we2_ref[...], preferred_element_type=jnp.float32)


def _embed(ea, we1, we2):
    fn = pl.pallas_call(
        _embed_body,
        grid=(E // EB,),
        in_specs=[pl.BlockSpec((EB, 16), lambda i: (i, 0)),
                  pl.BlockSpec((16, D), lambda i: (0, 0)),
                  pl.BlockSpec((16, D), lambda i: (0, 0))],
        out_specs=[pl.BlockSpec((EB, D), lambda i: (i, 0))] * 2,
        out_shape=[jax.ShapeDtypeStruct((E, D), jnp.float32)] * 2,
    )
    return fn(ea, we1, we2)


def _mid_body(pn_ref, pd_ref, s1_ref, r_ref, wq_ref, bq_ref, wk_ref,
              bk_ref, wv_ref, bv_ref, ws_ref, bs_ref,
              q_ref, k_ref, v_ref, s_ref, *, qscale):
    num = pn_ref[...]
    den = jnp.dot(pd_ref[...], r_ref[...], preferred_element_type=jnp.float32)
    h = num / (den + 1e-16) + s1_ref[...]
    h = jnp.where(h >= 0.0, h, 0.01 * h)
    dot = lambda w: jnp.dot(h, w[...], preferred_element_type=jnp.float32)
    q_ref[...] = (dot(wq_ref) + bq_ref[...]) * qscale
    k_ref[...] = dot(wk_ref) + bk_ref[...]
    v_ref[...] = dot(wv_ref) + bv_ref[...]
    s_ref[...] = dot(ws_ref) + bs_ref[...]


def _mid(pn, pd, s1, rmat, wq, bq, wk, bk, wv, bv, ws, bs, qscale):
    wspec = pl.BlockSpec((D, D), lambda i: (0, 0))
    bspec = pl.BlockSpec((1, D), lambda i: (0, 0))
    rspec = pl.BlockSpec((RN, D), lambda i: (i, 0))
    fn = pl.pallas_call(
        functools.partial(_mid_body, qscale=qscale),
        grid=(N // RN,),
        in_specs=[rspec, rspec, rspec, wspec,
                  wspec, bspec, wspec, bspec, wspec, bspec, wspec, bspec],
        out_specs=[rspec, rspec, rspec, rspec],
        out_shape=[jax.ShapeDtypeStruct((N, D), jnp.float32)] * 4,
    )
    return fn(pn, pd, s1, rmat, wq, bq.reshape(1, D), wk,
              bk.reshape(1, D), wv, bv.reshape(1, D), ws, bs.reshape(1, D))


def _final_body(pn_ref, pd_ref, s2_ref, r_ref, o_ref):
    den = jnp.dot(pd_ref[...], r_ref[...], preferred_element_type=jnp.float32)
    o_ref[...] = pn_ref[...] / (den + 1e-16) + s2_ref[...]


def _final(pn, pd, s2, rmat):
    rspec = pl.BlockSpec((RN, D), lambda i: (i, 0))
    fn = pl.pallas_call(
        _final_body,
        grid=(N // RN,),
        in_specs=[rspec, rspec, rspec,
                  pl.BlockSpec((D, D), lambda i: (0, 0))],
        out_specs=rspec,
        out_shape=jax.ShapeDtypeStruct((N, D), jnp.float32),
    )
    return fn(pn, pd, s2, rmat)


def _make_edge_kernel(heads):
    mesh = plsc.VectorSubcoreMesh(core_axis_name="c", subcore_axis_name="s")

    @functools.partial(
        pl.kernel,
        out_type=[jax.ShapeDtypeStruct((N, D), jnp.float32),
                  jax.ShapeDtypeStruct((N, D), jnp.float32)],
        mesh=mesh,
        compiler_params=pltpu.CompilerParams(use_tc_tiling_on_sc=False,
                                             needs_layout_passes=False),
        scratch_types=[
            pltpu.VMEM((B,), jnp.int32),
            pltpu.VMEM((B,), jnp.int32),
            pltpu.VMEM((B,), jnp.int32),
            pltpu.VMEM((B, D), jnp.float32),
            pltpu.VMEM((B, D), jnp.float32),
            pltpu.VMEM((B, D), jnp.float32),
            pltpu.VMEM((B, D), jnp.float32),
            pltpu.VMEM((B, D), jnp.float32),
            pltpu.VMEM((B, DW), jnp.float32),
            pltpu.VMEM((RB, D), jnp.float32),
            pltpu.VMEM((RB, DW), jnp.float32),
            pltpu.VMEM((RB, D), jnp.float32),
            pltpu.VMEM_SHARED((NH, D), jnp.float32),
            pltpu.VMEM_SHARED((NH, DW), jnp.float32),
        ],
    )
    def run(q_hbm, k_hbm, v_hbm, e_hbm, src_hbm, dst_hbm,
            on_hbm, od_hbm,
            si, di, dil, qr, kr, vr, er, mr, db, stg, stgd, pad,
            acc_n, acc_d):
        c = lax.axis_index("c")
        s = lax.axis_index("s")
        zero16 = jnp.zeros((16,), jnp.float32)

        @pl.loop(0, RB)
        def _(i):
            for j in range(D // 16):
                stg[i, pl.ds(j * 16, 16)] = zero16
                pad[i, pl.ds(j * 16, 16)] = zero16
            stgd[i, pl.ds(0, 16)] = zero16

        @pl.loop(0, B)
        def _(i):
            db[i, pl.ds(0, 16)] = zero16

        @pl.loop(s, NCH, step=NS)
        def _(j):
            pltpu.sync_copy(stg, acc_n.at[pl.ds(j * RB, RB)])
            pltpu.sync_copy(stgd, acc_d.at[pl.ds(j * RB, RB)])

        plsc.subcore_barrier()

        base = s * EW
        lo = c * NH

        @pl.loop(0, NB)
        def _(b):
            e0 = base + b * B
            pltpu.sync_copy(src_hbm.at[pl.ds(e0, B)], si)
            pltpu.sync_copy(dst_hbm.at[pl.ds(e0, B)], di)
            pltpu.sync_copy(q_hbm.at[di], qr)
            pltpu.sync_copy(k_hbm.at[si], kr)
            pltpu.sync_copy(v_hbm.at[si], vr)
            pltpu.sync_copy(e_hbm.at[pl.ds(e0, B)], er)

            @pl.loop(0, B // 16)
            def _(g):
                lanes = lax.iota(jnp.int32, 16)
                rows = g * 16 + lanes
                col = [jnp.full((16,), t, jnp.int32) for t in range(D)]
                dv = di[pl.ds(g * 16, 16)] - lo
                keep = (dv >= 0) & (dv < NH)
                dil[pl.ds(g * 16, 16)] = jnp.minimum(
                    jnp.maximum(dv, 0), NH - 1)
                if heads > 1:
                    cph = D // heads
                    for h in range(heads):
                        a = None
                        ve = []
                        for t in range(cph):
                            ci = col[h * cph + t]
                            qv = plsc.load_gather(qr, [rows, ci])
                            kv = plsc.load_gather(kr, [rows, ci])
                            vv = plsc.load_gather(vr, [rows, ci])
                            ev = plsc.load_gather(er, [rows, ci])
                            term = qv * (kv + ev)
                            a = term if a is None else a + term
                            ve.append(vv + ev)
                        p = jnp.where(keep, jnp.exp(a), 0.0)
                        plsc.store_scatter(db, [rows, col[h]], p)
                        for t in range(cph):
                            plsc.store_scatter(mr, [rows, col[h * cph + t]],
                                               p * ve[t])
                else:
                    a = None
                    for t in range(D):
                        ci = col[t]
                        qv = plsc.load_gather(qr, [rows, ci])
                        kv = plsc.load_gather(kr, [rows, ci])
                        ev = plsc.load_gather(er, [rows, ci])
                        term = qv * (kv + ev)
                        a = term if a is None else a + term
                    p = jnp.where(keep, jnp.exp(a), 0.0)
                    plsc.store_scatter(db, [rows, col[0]], p)
                    for t in range(D):
                        ci = col[t]
                        vv = plsc.load_gather(vr, [rows, ci])
                        ev = plsc.load_gather(er, [rows, ci])
                        plsc.store_scatter(mr, [rows, ci], p * (vv + ev))

            pltpu.sync_copy(mr, acc_n.at[dil], add=True)
            pltpu.sync_copy(db, acc_d.at[dil], add=True)

        plsc.subcore_barrier()

        @pl.loop(s, NCH, step=NS)
        def _(j):
            r0 = j * RB
            pltpu.sync_copy(acc_n.at[pl.ds(r0, RB)], stg)
            pltpu.sync_copy(stg, on_hbm.at[pl.ds(lo + r0, RB)])
            pltpu.sync_copy(acc_d.at[pl.ds(r0, RB)], stgd)

            @pl.loop(0, RB)
            def _(i):
                pad[i, pl.ds(0, 16)] = stgd[i, pl.ds(0, 16)]

            pltpu.sync_copy(pad, od_hbm.at[pl.ds(lo + r0, RB)])

    return run


_edge8 = _make_edge_kernel(8)
_edge1 = _make_edge_kernel(1)


def _rep_mat(heads):
    rmat = np.zeros((D, D), np.float32)
    cph = D // heads
    for h in range(heads):
        rmat[h, h * cph:(h + 1) * cph] = 1.0
    return jnp.asarray(rmat)


def kernel(x, ei, ea,
           Wq1, bq1, Wk1, bk1, Wv1, bv1, We1, Ws1, bs1,
           Wq2, bq2, Wk2, bk2, Wv2, bv2, We2, Ws2, bs2):
    src = ei[0]
    dst = ei[1]
    q1, k1, v1, s1 = _proj(x, Wq1, bq1, Wk1, bk1, Wv1, bv1, Ws1, bs1,
                           1.0 / math.sqrt(16.0))
    e1, e2 = _embed(ea, We1, We2)
    pn1, pd1 = _edge8(q1, k1, v1, e1, src, dst)
    q2, k2, v2, s2 = _mid(pn1, pd1, s1,
                          _rep_mat(8), Wq2, bq2, Wk2, bk2,
                          Wv2, bv2, Ws2, bs2, 1.0 / math.sqrt(128.0))
    pn2, pd2 = _edge1(q2, k2, v2, e2, src, dst)
    return _final(pn2, pd2, s2, _rep_mat(1))


# speedup vs baseline: 1.3398x; 1.3398x over previous
import functools
import math

import jax
import jax.numpy as jnp
import numpy as np
from jax import lax
from jax.experimental import pallas as pl
from jax.experimental.pallas import tpu as pltpu
from jax.experimental.pallas import tpu_sc as plsc

N = 10000
E = 320000
D = 128
DW = 16
NC, NS = 2, 16
NH = N // NC
EW = E // NS
B = 32
NB = EW // B
SEG = 5
EWS = EW // SEG
NBS = EWS // B
RB = 100
NCH = NH // RB
RN = 2000
EB = 4000


def _proj_body(x_ref, wq_ref, bq_ref, wk_ref, bk_ref, wv_ref, bv_ref,
               ws_ref, bs_ref, q_ref, k_ref, v_ref, s_ref, *, qscale):
    xb = x_ref[...]
    dot = lambda w: jnp.dot(xb, w[...], preferred_element_type=jnp.float32)
    q_ref[...] = (dot(wq_ref) + bq_ref[...]) * qscale
    k_ref[...] = dot(wk_ref) + bk_ref[...]
    v_ref[...] = dot(wv_ref) + bv_ref[...]
    s_ref[...] = dot(ws_ref) + bs_ref[...]


def _proj(x, wq, bq, wk, bk, wv, bv, ws, bs, qscale):
    n = x.shape[0]
    wspec = pl.BlockSpec((D, D), lambda i: (0, 0))
    bspec = pl.BlockSpec((1, D), lambda i: (0, 0))
    rspec = pl.BlockSpec((RN, D), lambda i: (i, 0))
    fn = pl.pallas_call(
        functools.partial(_proj_body, qscale=qscale),
        grid=(n // RN,),
        in_specs=[rspec, wspec, bspec, wspec, bspec, wspec, bspec, wspec,
                  bspec],
        out_specs=[rspec, rspec, rspec, rspec],
        out_shape=[jax.ShapeDtypeStruct((n, D), jnp.float32)] * 4,
    )
    return fn(x, wq, bq.reshape(1, D), wk, bk.reshape(1, D),
              wv, bv.reshape(1, D), ws, bs.reshape(1, D))


def _embed_body(ea_ref, we1_ref, we2_ref, e1_ref, e2_ref):
    eab = ea_ref[...]
    e1_ref[...] = jnp.dot(eab, we1_ref[...], preferred_element_type=jnp.float32)
    e2_ref[...] = jnp.dot(eab, we2_ref[...], preferred_element_type=jnp.float32)


def _embed(ea, we1, we2):
    fn = pl.pallas_call(
        _embed_body,
        grid=(E // EB,),
        in_specs=[pl.BlockSpec((EB, 16), lambda i: (i, 0)),
                  pl.BlockSpec((16, D), lambda i: (0, 0)),
                  pl.BlockSpec((16, D), lambda i: (0, 0))],
        out_specs=[pl.BlockSpec((EB, D), lambda i: (i, 0))] * 2,
        out_shape=[jax.ShapeDtypeStruct((E, D), jnp.float32)] * 2,
    )
    return fn(ea, we1, we2)


def _mid_body(pn_ref, pd_ref, s1_ref, r_ref, wq_ref, bq_ref, wk_ref,
              bk_ref, wv_ref, bv_ref, ws_ref, bs_ref,
              q_ref, k_ref, v_ref, s_ref, *, qscale):
    num = pn_ref[...]
    den = jnp.dot(pd_ref[...], r_ref[...], preferred_element_type=jnp.float32)
    h = num / (den + 1e-16) + s1_ref[...]
    h = jnp.where(h >= 0.0, h, 0.01 * h)
    dot = lambda w: jnp.dot(h, w[...], preferred_element_type=jnp.float32)
    q_ref[...] = (dot(wq_ref) + bq_ref[...]) * qscale
    k_ref[...] = dot(wk_ref) + bk_ref[...]
    v_ref[...] = dot(wv_ref) + bv_ref[...]
    s_ref[...] = dot(ws_ref) + bs_ref[...]


def _mid(pn, pd, s1, rmat, wq, bq, wk, bk, wv, bv, ws, bs, qscale):
    wspec = pl.BlockSpec((D, D), lambda i: (0, 0))
    bspec = pl.BlockSpec((1, D), lambda i: (0, 0))
    rspec = pl.BlockSpec((RN, D), lambda i: (i, 0))
    fn = pl.pallas_call(
        functools.partial(_mid_body, qscale=qscale),
        grid=(N // RN,),
        in_specs=[rspec, rspec, rspec, wspec,
                  wspec, bspec, wspec, bspec, wspec, bspec, wspec, bspec],
        out_specs=[rspec, rspec, rspec, rspec],
        out_shape=[jax.ShapeDtypeStruct((N, D), jnp.float32)] * 4,
    )
    return fn(pn, pd, s1, rmat, wq, bq.reshape(1, D), wk,
              bk.reshape(1, D), wv, bv.reshape(1, D), ws, bs.reshape(1, D))


def _final_body(pn_ref, pd_ref, s2_ref, r_ref, o_ref):
    den = jnp.dot(pd_ref[...], r_ref[...], preferred_element_type=jnp.float32)
    o_ref[...] = pn_ref[...] / (den + 1e-16) + s2_ref[...]


def _final(pn, pd, s2, rmat):
    rspec = pl.BlockSpec((RN, D), lambda i: (i, 0))
    fn = pl.pallas_call(
        _final_body,
        grid=(N // RN,),
        in_specs=[rspec, rspec, rspec,
                  pl.BlockSpec((D, D), lambda i: (0, 0))],
        out_specs=rspec,
        out_shape=jax.ShapeDtypeStruct((N, D), jnp.float32),
    )
    return fn(pn, pd, s2, rmat)


def _make_edge_kernel(heads):
    assert NBS % 2 == 1
    mesh = plsc.VectorSubcoreMesh(core_axis_name="c", subcore_axis_name="s")

    @functools.partial(
        pl.kernel,
        out_type=[jax.ShapeDtypeStruct((N, D), jnp.float32),
                  jax.ShapeDtypeStruct((N, D), jnp.float32)],
        mesh=mesh,
        compiler_params=pltpu.CompilerParams(use_tc_tiling_on_sc=False,
                                             needs_layout_passes=False),
        scratch_types=[
            pltpu.VMEM((EWS,), jnp.int32),
            pltpu.VMEM((EWS,), jnp.int32),
            pltpu.VMEM((B,), jnp.int32),
            pltpu.VMEM((B,), jnp.int32),
            pltpu.VMEM((B, D), jnp.float32),
            pltpu.VMEM((B, D), jnp.float32),
            pltpu.VMEM((B, D), jnp.float32),
            pltpu.VMEM((B, D), jnp.float32),
            pltpu.VMEM((B, D), jnp.float32),
            pltpu.VMEM((B, D), jnp.float32),
            pltpu.VMEM((B, D), jnp.float32),
            pltpu.VMEM((B, D), jnp.float32),
            pltpu.VMEM((B, D), jnp.float32),
            pltpu.VMEM((B, D), jnp.float32),
            pltpu.VMEM((B, DW), jnp.float32),
            pltpu.VMEM((B, DW), jnp.float32),
            pltpu.VMEM((RB, D), jnp.float32),
            pltpu.VMEM((RB, DW), jnp.float32),
            pltpu.VMEM_SHARED((NH, D), jnp.float32),
            pltpu.VMEM_SHARED((NH, DW), jnp.float32),
            pltpu.SemaphoreType.DMA,
            pltpu.SemaphoreType.DMA,
            pltpu.SemaphoreType.DMA,
            pltpu.SemaphoreType.DMA,
        ],
    )
    def run(q_hbm, k_hbm, v_hbm, e_hbm, src_hbm, dst_hbm,
            on_hbm, od_hbm,
            sia, dia, dil0, dil1, qr0, qr1, kr0, kr1, vr0, vr1, er0, er1,
            mr0, mr1, db0, db1, stg, stgd,
            acc_n, acc_d, gsem0, gsem1, ssem0, ssem1):
        dil = [dil0, dil1]
        qr = [qr0, qr1]
        kr = [kr0, kr1]
        vr = [vr0, vr1]
        er = [er0, er1]
        mr = [mr0, mr1]
        db = [db0, db1]
        gsem = [gsem0, gsem1]
        ssem = [ssem0, ssem1]
        c = lax.axis_index("c")
        s = lax.axis_index("s")
        zero16 = jnp.zeros((16,), jnp.float32)

        @pl.loop(0, RB)
        def _(i):
            for j in range(D // 16):
                stg[i, pl.ds(j * 16, 16)] = zero16
            stgd[i, pl.ds(0, 16)] = zero16

        @pl.loop(0, B)
        def _(i):
            db0[i, pl.ds(0, 16)] = zero16
            db1[i, pl.ds(0, 16)] = zero16

        @pl.loop(s, NCH, step=NS)
        def _(j):
            pltpu.sync_copy(stg, acc_n.at[pl.ds(j * RB, RB)])
            pltpu.sync_copy(stgd, acc_d.at[pl.ds(j * RB, RB)])

        plsc.subcore_barrier()

        base = s * EW
        lo = c * NH

        def issue_gathers(sb, lb, u):
            pltpu.async_copy(q_hbm.at[dia.at[pl.ds(lb * B, B)]], qr[u],
                             gsem[u])
            pltpu.async_copy(k_hbm.at[sia.at[pl.ds(lb * B, B)]], kr[u],
                             gsem[u])
            pltpu.async_copy(v_hbm.at[sia.at[pl.ds(lb * B, B)]], vr[u],
                             gsem[u])
            pltpu.async_copy(e_hbm.at[pl.ds(sb + lb * B, B)], er[u],
                             gsem[u])

        def drain_gathers(sb, lb, u):
            pltpu.make_async_copy(q_hbm.at[dia.at[pl.ds(lb * B, B)]], qr[u],
                                  gsem[u]).wait()
            pltpu.make_async_copy(k_hbm.at[sia.at[pl.ds(lb * B, B)]], kr[u],
                                  gsem[u]).wait()
            pltpu.make_async_copy(v_hbm.at[sia.at[pl.ds(lb * B, B)]], vr[u],
                                  gsem[u]).wait()
            pltpu.make_async_copy(e_hbm.at[pl.ds(sb + lb * B, B)], er[u],
                                  gsem[u]).wait()

        def drain_scatters(u):
            pltpu.make_async_copy(mr[u], acc_n.at[dil[u]], ssem[u]).wait()
            pltpu.make_async_copy(db[u], acc_d.at[dil[u]], ssem[u]).wait()

        def compute_block(lb, u):
            @pl.loop(0, B // 16)
            def _(g):
                lanes = lax.iota(jnp.int32, 16)
                rows = g * 16 + lanes
                col = [jnp.full((16,), t, jnp.int32) for t in range(D)]
                dv = dia[pl.ds(lb * B + g * 16, 16)] - lo
                keep = (dv >= 0) & (dv < NH)
                dil[u][pl.ds(g * 16, 16)] = jnp.minimum(
                    jnp.maximum(dv, 0), NH - 1)
                if heads > 1:
                    cph = D // heads
                    for h in range(heads):
                        a = None
                        ve = []
                        for t in range(cph):
                            ci = col[h * cph + t]
                            qv = plsc.load_gather(qr[u], [rows, ci])
                            kv = plsc.load_gather(kr[u], [rows, ci])
                            vv = plsc.load_gather(vr[u], [rows, ci])
                            ev = plsc.load_gather(er[u], [rows, ci])
                            term = qv * (kv + ev)
                            a = term if a is None else a + term
                            ve.append(vv + ev)
                        p = jnp.where(keep, jnp.exp(a), 0.0)
                        plsc.store_scatter(db[u], [rows, col[h]], p)
                        for t in range(cph):
                            plsc.store_scatter(mr[u],
                                               [rows, col[h * cph + t]],
                                               p * ve[t])
                else:
                    a = None
                    for t in range(D):
                        ci = col[t]
                        qv = plsc.load_gather(qr[u], [rows, ci])
                        kv = plsc.load_gather(kr[u], [rows, ci])
                        ev = plsc.load_gather(er[u], [rows, ci])
                        term = qv * (kv + ev)
                        a = term if a is None else a + term
                    p = jnp.where(keep, jnp.exp(a), 0.0)
                    plsc.store_scatter(db[u], [rows, col[0]], p)
                    for t in range(D):
                        ci = col[t]
                        vv = plsc.load_gather(vr[u], [rows, ci])
                        ev = plsc.load_gather(er[u], [rows, ci])
                        plsc.store_scatter(mr[u], [rows, ci], p * (vv + ev))

        def emit_block(sb, lb, u, issue_next):
            @pl.when(lb >= 2)
            def _():
                drain_scatters(u)

            drain_gathers(sb, lb, u)
            if issue_next:
                issue_gathers(sb, lb + 1, 1 - u)
            compute_block(lb, u)
            pltpu.async_copy(mr[u], acc_n.at[dil[u]], ssem[u], add=True)
            pltpu.async_copy(db[u], acc_d.at[dil[u]], ssem[u], add=True)

        @pl.loop(0, SEG)
        def _(seg):
            sb = base + seg * EWS
            pltpu.sync_copy(src_hbm.at[pl.ds(sb, EWS)], sia)
            pltpu.sync_copy(dst_hbm.at[pl.ds(sb, EWS)], dia)
            issue_gathers(sb, 0, 0)

            @pl.loop(0, (NBS - 1) // 2)
            def _(j):
                emit_block(sb, 2 * j, 0, True)
                emit_block(sb, 2 * j + 1, 1, True)

            emit_block(sb, NBS - 1, (NBS - 1) % 2, False)
            drain_scatters((NBS - 2) % 2)
            drain_scatters((NBS - 1) % 2)

        plsc.subcore_barrier()

        @pl.loop(s, NCH, step=NS)
        def _(j):
            r0 = j * RB
            pltpu.sync_copy(acc_n.at[pl.ds(r0, RB)], stg)
            pltpu.sync_copy(stg, on_hbm.at[pl.ds(lo + r0, RB)])
            pltpu.sync_copy(acc_d.at[pl.ds(r0, RB)], stgd)

            @pl.loop(0, RB)
            def _(i):
                stg[i, pl.ds(0, 16)] = stgd[i, pl.ds(0, 16)]
                for j in range(1, D // 16):
                    stg[i, pl.ds(j * 16, 16)] = zero16

            pltpu.sync_copy(stg, od_hbm.at[pl.ds(lo + r0, RB)])

    return run


_edge8 = _make_edge_kernel(8)
_edge1 = _make_edge_kernel(1)


def _rep_mat(heads):
    rmat = np.zeros((D, D), np.float32)
    cph = D // heads
    for h in range(heads):
        rmat[h, h * cph:(h + 1) * cph] = 1.0
    return jnp.asarray(rmat)


def kernel(x, ei, ea,
           Wq1, bq1, Wk1, bk1, Wv1, bv1, We1, Ws1, bs1,
           Wq2, bq2, Wk2, bk2, Wv2, bv2, We2, Ws2, bs2):
    src = ei[0]
    dst = ei[1]
    q1, k1, v1, s1 = _proj(x, Wq1, bq1, Wk1, bk1, Wv1, bv1, Ws1, bs1,
                           1.0 / math.sqrt(16.0))
    e1, e2 = _embed(ea, We1, We2)
    pn1, pd1 = _edge8(q1, k1, v1, e1, src, dst)
    q2, k2, v2, s2 = _mid(pn1, pd1, s1,
                          _rep_mat(8), Wq2, bq2, Wk2, bk2,
                          Wv2, bv2, Ws2, bs2, 1.0 / math.sqrt(128.0))
    pn2, pd2 = _edge1(q2, k2, v2, e2, src, dst)
    return _final(pn2, pd2, s2, _rep_mat(1))


# speedup vs baseline: 2.4545x; 1.8320x over previous
import functools
import math

import jax
import jax.numpy as jnp
import numpy as np
from jax import lax
from jax.experimental import pallas as pl
from jax.experimental.pallas import tpu as pltpu
from jax.experimental.pallas import tpu_sc as plsc

N = 10000
E = 320000
D = 128
DW = 16
NC, NS = 2, 16
EW = E // (NC * NS)
B = 16
NB = EW // B
SEG = 5
EWS = EW // SEG
NBS = EWS // B
RB = 50
NCH = N // RB
RN = 2000
EB = 4000


def _proj_body(x_ref, wq_ref, bq_ref, wk_ref, bk_ref, wv_ref, bv_ref,
               ws_ref, bs_ref, q_ref, k_ref, v_ref, s_ref, *, qscale):
    xb = x_ref[...]
    dot = lambda w: jnp.dot(xb, w[...], preferred_element_type=jnp.float32)
    q_ref[...] = (dot(wq_ref) + bq_ref[...]) * qscale
    k_ref[...] = dot(wk_ref) + bk_ref[...]
    v_ref[...] = dot(wv_ref) + bv_ref[...]
    s_ref[...] = dot(ws_ref) + bs_ref[...]


def _proj(x, wq, bq, wk, bk, wv, bv, ws, bs, qscale):
    n = x.shape[0]
    wspec = pl.BlockSpec((D, D), lambda i: (0, 0))
    bspec = pl.BlockSpec((1, D), lambda i: (0, 0))
    rspec = pl.BlockSpec((RN, D), lambda i: (i, 0))
    fn = pl.pallas_call(
        functools.partial(_proj_body, qscale=qscale),
        grid=(n // RN,),
        in_specs=[rspec, wspec, bspec, wspec, bspec, wspec, bspec, wspec,
                  bspec],
        out_specs=[rspec, rspec, rspec, rspec],
        out_shape=[jax.ShapeDtypeStruct((n, D), jnp.float32)] * 4,
    )
    return fn(x, wq, bq.reshape(1, D), wk, bk.reshape(1, D),
              wv, bv.reshape(1, D), ws, bs.reshape(1, D))


def _embed_body(ea_ref, we1_ref, we2_ref, e1_ref, e2_ref):
    eab = ea_ref[...]
    e1_ref[...] = jnp.dot(eab, we1_ref[...], preferred_element_type=jnp.float32)
    e2_ref[...] = jnp.dot(eab, we2_ref[...], preferred_element_type=jnp.float32)


def _embed(ea, we1, we2):
    fn = pl.pallas_call(
        _embed_body,
        grid=(E // EB,),
        in_specs=[pl.BlockSpec((EB, 16), lambda i: (i, 0)),
                  pl.BlockSpec((16, D), lambda i: (0, 0)),
                  pl.BlockSpec((16, D), lambda i: (0, 0))],
        out_specs=[pl.BlockSpec((EB, D), lambda i: (i, 0))] * 2,
        out_shape=[jax.ShapeDtypeStruct((E, D), jnp.float32)] * 2,
    )
    return fn(ea, we1, we2)


def _mid_body(pn0_ref, pn1_ref, pd0_ref, pd1_ref, s1_ref, r_ref, wq_ref,
              bq_ref, wk_ref, bk_ref, wv_ref, bv_ref, ws_ref, bs_ref,
              q_ref, k_ref, v_ref, s_ref, *, qscale):
    num = pn0_ref[...] + pn1_ref[...]
    den = jnp.dot(pd0_ref[...] + pd1_ref[...], r_ref[...],
                  preferred_element_type=jnp.float32)
    h = num / (den + 1e-16) + s1_ref[...]
    h = jnp.where(h >= 0.0, h, 0.01 * h)
    dot = lambda w: jnp.dot(h, w[...], preferred_element_type=jnp.float32)
    q_ref[...] = (dot(wq_ref) + bq_ref[...]) * qscale
    k_ref[...] = dot(wk_ref) + bk_ref[...]
    v_ref[...] = dot(wv_ref) + bv_ref[...]
    s_ref[...] = dot(ws_ref) + bs_ref[...]


def _mid(pn, pd, s1, rmat, wq, bq, wk, bk, wv, bv, ws, bs, qscale):
    wspec = pl.BlockSpec((D, D), lambda i: (0, 0))
    bspec = pl.BlockSpec((1, D), lambda i: (0, 0))
    rspec = pl.BlockSpec((RN, D), lambda i: (i, 0))
    fn = pl.pallas_call(
        functools.partial(_mid_body, qscale=qscale),
        grid=(N // RN,),
        in_specs=[rspec, rspec, rspec, rspec, rspec, wspec,
                  wspec, bspec, wspec, bspec, wspec, bspec, wspec, bspec],
        out_specs=[rspec, rspec, rspec, rspec],
        out_shape=[jax.ShapeDtypeStruct((N, D), jnp.float32)] * 4,
    )
    return fn(pn[:N], pn[N:], pd[:N], pd[N:], s1, rmat, wq, bq.reshape(1, D),
              wk, bk.reshape(1, D), wv, bv.reshape(1, D), ws,
              bs.reshape(1, D))


def _final_body(pn0_ref, pn1_ref, pd0_ref, pd1_ref, s2_ref, r_ref, o_ref):
    den = jnp.dot(pd0_ref[...] + pd1_ref[...], r_ref[...],
                  preferred_element_type=jnp.float32)
    o_ref[...] = (pn0_ref[...] + pn1_ref[...]) / (den + 1e-16) + s2_ref[...]


def _final(pn, pd, s2, rmat):
    rspec = pl.BlockSpec((RN, D), lambda i: (i, 0))
    fn = pl.pallas_call(
        _final_body,
        grid=(N // RN,),
        in_specs=[rspec, rspec, rspec, rspec, rspec,
                  pl.BlockSpec((D, D), lambda i: (0, 0))],
        out_specs=rspec,
        out_shape=jax.ShapeDtypeStruct((N, D), jnp.float32),
    )
    return fn(pn[:N], pn[N:], pd[:N], pd[N:], s2, rmat)


def _make_edge_kernel(heads):
    assert NBS % 2 == 1
    mesh = plsc.VectorSubcoreMesh(core_axis_name="c", subcore_axis_name="s")

    @functools.partial(
        pl.kernel,
        out_type=[jax.ShapeDtypeStruct((NC * N, D), jnp.float32),
                  jax.ShapeDtypeStruct((NC * N, D), jnp.float32)],
        mesh=mesh,
        compiler_params=pltpu.CompilerParams(use_tc_tiling_on_sc=False,
                                             needs_layout_passes=False),
        scratch_types=[
            pltpu.VMEM((EWS,), jnp.int32),
            pltpu.VMEM((EWS,), jnp.int32),
            pltpu.VMEM((B,), jnp.int32),
            pltpu.VMEM((B,), jnp.int32),
            pltpu.VMEM((B, D), jnp.float32),
            pltpu.VMEM((B, D), jnp.float32),
            pltpu.VMEM((B, D), jnp.float32),
            pltpu.VMEM((B, D), jnp.float32),
            pltpu.VMEM((B, D), jnp.float32),
            pltpu.VMEM((B, D), jnp.float32),
            pltpu.VMEM((B, D), jnp.float32),
            pltpu.VMEM((B, D), jnp.float32),
            pltpu.VMEM((B, D), jnp.float32),
            pltpu.VMEM((B, D), jnp.float32),
            pltpu.VMEM((B, DW), jnp.float32),
            pltpu.VMEM((B, DW), jnp.float32),
            pltpu.VMEM((RB, D), jnp.float32),
            pltpu.VMEM((RB, DW), jnp.float32),
            pltpu.VMEM_SHARED((N, D), jnp.float32),
            pltpu.VMEM_SHARED((N, DW), jnp.float32),
            pltpu.SemaphoreType.DMA,
            pltpu.SemaphoreType.DMA,
            pltpu.SemaphoreType.DMA,
            pltpu.SemaphoreType.DMA,
        ],
    )
    def run(q_hbm, k_hbm, v_hbm, e_hbm, src_hbm, dst_hbm,
            on_hbm, od_hbm,
            sia, dia, dil0, dil1, qr0, qr1, kr0, kr1, vr0, vr1, er0, er1,
            mr0, mr1, db0, db1, stg, stgd,
            acc_n, acc_d, gsem0, gsem1, ssem0, ssem1):
        dil = [dil0, dil1]
        qr = [qr0, qr1]
        kr = [kr0, kr1]
        vr = [vr0, vr1]
        er = [er0, er1]
        mr = [mr0, mr1]
        db = [db0, db1]
        gsem = [gsem0, gsem1]
        ssem = [ssem0, ssem1]
        c = lax.axis_index("c")
        s = lax.axis_index("s")
        zero16 = jnp.zeros((16,), jnp.float32)

        @pl.loop(0, RB)
        def _(i):
            for j in range(D // 16):
                stg[i, pl.ds(j * 16, 16)] = zero16
            stgd[i, pl.ds(0, 16)] = zero16

        @pl.loop(0, B)
        def _(i):
            db0[i, pl.ds(0, 16)] = zero16
            db1[i, pl.ds(0, 16)] = zero16

        @pl.loop(s, NCH, step=NS)
        def _(j):
            pltpu.sync_copy(stg, acc_n.at[pl.ds(j * RB, RB)])
            pltpu.sync_copy(stgd, acc_d.at[pl.ds(j * RB, RB)])

        plsc.subcore_barrier()

        base = (c * NS + s) * EW
        lo = c * N

        def issue_gathers(sb, lb, u):
            pltpu.async_copy(q_hbm.at[dia.at[pl.ds(lb * B, B)]], qr[u],
                             gsem[u])
            pltpu.async_copy(k_hbm.at[sia.at[pl.ds(lb * B, B)]], kr[u],
                             gsem[u])
            pltpu.async_copy(v_hbm.at[sia.at[pl.ds(lb * B, B)]], vr[u],
                             gsem[u])
            pltpu.async_copy(e_hbm.at[pl.ds(sb + lb * B, B)], er[u],
                             gsem[u])

        def drain_gathers(sb, lb, u):
            pltpu.make_async_copy(q_hbm.at[dia.at[pl.ds(lb * B, B)]], qr[u],
                                  gsem[u]).wait()
            pltpu.make_async_copy(k_hbm.at[sia.at[pl.ds(lb * B, B)]], kr[u],
                                  gsem[u]).wait()
            pltpu.make_async_copy(v_hbm.at[sia.at[pl.ds(lb * B, B)]], vr[u],
                                  gsem[u]).wait()
            pltpu.make_async_copy(e_hbm.at[pl.ds(sb + lb * B, B)], er[u],
                                  gsem[u]).wait()

        def drain_scatters(u):
            pltpu.make_async_copy(mr[u], acc_n.at[dil[u]], ssem[u]).wait()
            pltpu.make_async_copy(db[u], acc_d.at[dil[u]], ssem[u]).wait()

        def compute_block(lb, u):
            @pl.loop(0, B // 16)
            def _(g):
                lanes = lax.iota(jnp.int32, 16)
                rows = g * 16 + lanes
                col = [jnp.full((16,), t, jnp.int32) for t in range(D)]
                dil[u][pl.ds(g * 16, 16)] = dia[pl.ds(lb * B + g * 16, 16)]
                if heads > 1:
                    cph = D // heads
                    for h in range(heads):
                        a = None
                        ve = []
                        for t in range(cph):
                            ci = col[h * cph + t]
                            qv = plsc.load_gather(qr[u], [rows, ci])
                            kv = plsc.load_gather(kr[u], [rows, ci])
                            vv = plsc.load_gather(vr[u], [rows, ci])
                            ev = plsc.load_gather(er[u], [rows, ci])
                            term = qv * (kv + ev)
                            a = term if a is None else a + term
                            ve.append(vv + ev)
                        p = jnp.exp(a)
                        plsc.store_scatter(db[u], [rows, col[h]], p)
                        for t in range(cph):
                            plsc.store_scatter(mr[u],
                                               [rows, col[h * cph + t]],
                                               p * ve[t])
                else:
                    a = None
                    for t in range(D):
                        ci = col[t]
                        qv = plsc.load_gather(qr[u], [rows, ci])
                        kv = plsc.load_gather(kr[u], [rows, ci])
                        ev = plsc.load_gather(er[u], [rows, ci])
                        term = qv * (kv + ev)
                        a = term if a is None else a + term
                    p = jnp.exp(a)
                    plsc.store_scatter(db[u], [rows, col[0]], p)
                    for t in range(D):
                        ci = col[t]
                        vv = plsc.load_gather(vr[u], [rows, ci])
                        ev = plsc.load_gather(er[u], [rows, ci])
                        plsc.store_scatter(mr[u], [rows, ci], p * (vv + ev))

        def emit_block(sb, lb, u, issue_next):
            @pl.when(lb >= 2)
            def _():
                drain_scatters(u)

            drain_gathers(sb, lb, u)
            if issue_next:
                issue_gathers(sb, lb + 1, 1 - u)
            compute_block(lb, u)
            pltpu.async_copy(mr[u], acc_n.at[dil[u]], ssem[u], add=True)
            pltpu.async_copy(db[u], acc_d.at[dil[u]], ssem[u], add=True)

        @pl.loop(0, SEG)
        def _(seg):
            sb = base + seg * EWS
            pltpu.sync_copy(src_hbm.at[pl.ds(sb, EWS)], sia)
            pltpu.sync_copy(dst_hbm.at[pl.ds(sb, EWS)], dia)
            issue_gathers(sb, 0, 0)

            @pl.loop(0, (NBS - 1) // 2)
            def _(j):
                emit_block(sb, 2 * j, 0, True)
                emit_block(sb, 2 * j + 1, 1, True)

            emit_block(sb, NBS - 1, (NBS - 1) % 2, False)
            drain_scatters((NBS - 2) % 2)
            drain_scatters((NBS - 1) % 2)

        plsc.subcore_barrier()

        @pl.loop(s, NCH, step=NS)
        def _(j):
            r0 = j * RB
            pltpu.sync_copy(acc_n.at[pl.ds(r0, RB)], stg)
            pltpu.sync_copy(stg, on_hbm.at[pl.ds(lo + r0, RB)])
            pltpu.sync_copy(acc_d.at[pl.ds(r0, RB)], stgd)

            @pl.loop(0, RB)
            def _(i):
                stg[i, pl.ds(0, 16)] = stgd[i, pl.ds(0, 16)]
                for j in range(1, D // 16):
                    stg[i, pl.ds(j * 16, 16)] = zero16

            pltpu.sync_copy(stg, od_hbm.at[pl.ds(lo + r0, RB)])

    return run


_edge8 = _make_edge_kernel(8)
_edge1 = _make_edge_kernel(1)


def _rep_mat(heads):
    rmat = np.zeros((D, D), np.float32)
    cph = D // heads
    for h in range(heads):
        rmat[h, h * cph:(h + 1) * cph] = 1.0
    return jnp.asarray(rmat)


def kernel(x, ei, ea,
           Wq1, bq1, Wk1, bk1, Wv1, bv1, We1, Ws1, bs1,
           Wq2, bq2, Wk2, bk2, Wv2, bv2, We2, Ws2, bs2):
    src = ei[0]
    dst = ei[1]
    q1, k1, v1, s1 = _proj(x, Wq1, bq1, Wk1, bk1, Wv1, bv1, Ws1, bs1,
                           1.0 / math.sqrt(16.0))
    e1, e2 = _embed(ea, We1, We2)
    pn1, pd1 = _edge8(q1, k1, v1, e1, src, dst)
    q2, k2, v2, s2 = _mid(pn1, pd1, s1,
                          _rep_mat(8), Wq2, bq2, Wk2, bk2,
                          Wv2, bv2, Ws2, bs2, 1.0 / math.sqrt(128.0))
    pn2, pd2 = _edge1(q2, k2, v2, e2, src, dst)
    return _final(pn2, pd2, s2, _rep_mat(1))
